# baseline (speedup 1.0000x reference)
import functools

import jax
import jax.numpy as jnp
from jax import lax
from jax.experimental import pallas as pl
from jax.experimental.pallas import tpu as pltpu
from jax.experimental.pallas import tpu_sc as plsc

VOCAB = 1_000_000
DIM = 32
BATCH = 16384
NUM_NEG = 5

NUM_CORES = 2
NUM_SUBCORES = 16
NUM_WORKERS = NUM_CORES * NUM_SUBCORES
BPW = BATCH // NUM_WORKERS
CHUNK = 128
N_CHUNKS = BPW // CHUNK
N_NEG_CHUNKS = BPW * NUM_NEG // CHUNK
NSLOT = 1 + NUM_NEG


def _sc_body(inp_idx_hbm, tgt_idx_hbm, neg_idx_hbm, emb_hbm, outw_hbm,
             logits_hbm,
             idx_i_v, idx_t_v, idx_n_v, inp_rows, tgt_rows, neg_rows,
             logit_v, sem):
    wid = lax.axis_index("s") * NUM_CORES + lax.axis_index("c")

    pltpu.sync_copy(inp_idx_hbm.at[pl.ds(wid * BPW, BPW)], idx_i_v)
    pltpu.sync_copy(tgt_idx_hbm.at[pl.ds(wid * BPW, BPW)], idx_t_v)
    pltpu.sync_copy(neg_idx_hbm.at[pl.ds(wid * BPW * NUM_NEG, BPW * NUM_NEG)],
                    idx_n_v)

    copies = []
    for j in range(N_CHUNKS):
        c = pltpu.make_async_copy(emb_hbm.at[idx_i_v.at[pl.ds(j * CHUNK, CHUNK)]],
                                  inp_rows.at[pl.ds(j * CHUNK, CHUNK)], sem)
        c.start()
        copies.append(c)
    for j in range(N_CHUNKS):
        c = pltpu.make_async_copy(outw_hbm.at[idx_t_v.at[pl.ds(j * CHUNK, CHUNK)]],
                                  tgt_rows.at[pl.ds(j * CHUNK, CHUNK)], sem)
        c.start()
        copies.append(c)
    for j in range(N_NEG_CHUNKS):
        c = pltpu.make_async_copy(outw_hbm.at[idx_n_v.at[pl.ds(j * CHUNK, CHUNK)]],
                                  neg_rows.at[pl.ds(j * CHUNK, CHUNK)], sem)
        c.start()
        copies.append(c)
    for c in copies:
        c.wait()

    lane = lax.iota(jnp.int32, 16)
    zeros = jnp.zeros((16,), jnp.float32)

    def group_body(g, carry):
        b0 = g * 16
        rows16 = b0 + lane
        rows_neg = rows16 * NUM_NEG

        def d_body(d, accs):
            dcol = jnp.zeros((16,), jnp.int32) + d
            iv = plsc.load_gather(inp_rows, [rows16, dcol])
            tv = plsc.load_gather(tgt_rows, [rows16, dcol])
            new = [accs[0] - iv * tv]
            for k in range(NUM_NEG):
                nv = plsc.load_gather(neg_rows, [rows_neg + k, dcol])
                new.append(accs[k + 1] + iv * nv)
            return tuple(new)

        accs = lax.fori_loop(0, DIM, d_body, (zeros,) * NSLOT)
        for s in range(NSLOT):
            logit_v[pl.ds(s * BPW + b0, 16)] = accs[s]
        return carry

    lax.fori_loop(0, BPW // 16, group_body, 0)

    pltpu.sync_copy(logit_v, logits_hbm.at[pl.ds(wid * NSLOT * BPW, NSLOT * BPW)])


def _tc_body(x_ref, o_ref):
    x = x_ref[...]
    o_ref[0, 0] = jnp.sum(jax.nn.softplus(x)) / BATCH


def kernel(input_idx, target_idx, neg_idx, emb_table, out_table):
    inp1d = input_idx.astype(jnp.int32).reshape(BATCH)
    tgt1d = target_idx.astype(jnp.int32).reshape(BATCH)
    neg1d = neg_idx.astype(jnp.int32).reshape(BATCH * NUM_NEG)

    mesh = plsc.VectorSubcoreMesh(core_axis_name="c", subcore_axis_name="s",
                                  num_cores=NUM_CORES,
                                  num_subcores=NUM_SUBCORES)
    sc_fn = pl.kernel(
        _sc_body,
        out_type=jax.ShapeDtypeStruct((NUM_WORKERS * NSLOT * BPW,),
                                      jnp.float32),
        mesh=mesh,
        compiler_params=pltpu.CompilerParams(needs_layout_passes=False,
                                             use_tc_tiling_on_sc=False),
        scratch_types=[
            pltpu.VMEM((BPW,), jnp.int32),
            pltpu.VMEM((BPW,), jnp.int32),
            pltpu.VMEM((BPW * NUM_NEG,), jnp.int32),
            pltpu.VMEM((BPW, DIM), jnp.float32),
            pltpu.VMEM((BPW, DIM), jnp.float32),
            pltpu.VMEM((BPW * NUM_NEG, DIM), jnp.float32),
            pltpu.VMEM((NSLOT * BPW,), jnp.float32),
            pltpu.SemaphoreType.DMA,
        ],
    )
    logits = sc_fn(inp1d, tgt1d, neg1d, emb_table, out_table)

    flat = logits.reshape(NUM_WORKERS * NSLOT * BPW // CHUNK, CHUNK)
    loss = pl.pallas_call(
        _tc_body,
        out_shape=jax.ShapeDtypeStruct((1, 1), jnp.float32),
        out_specs=pl.BlockSpec(memory_space=pltpu.SMEM),
    )(flat)
    return loss[0, 0]


# speedup vs baseline: 1.1889x; 1.1889x over previous
import jax
import jax.numpy as jnp
from jax import lax
from jax.experimental import pallas as pl
from jax.experimental.pallas import tpu as pltpu
from jax.experimental.pallas import tpu_sc as plsc

VOCAB = 1_000_000
DIM = 32
BATCH = 16384
NUM_NEG = 5

NUM_CORES = 2
NUM_SUBCORES = 16
NUM_WORKERS = NUM_CORES * NUM_SUBCORES
BPW = BATCH // NUM_WORKERS
CHUNK = 128
N_CHUNKS = BPW // CHUNK
N_NEG_CHUNKS = BPW * NUM_NEG // CHUNK
NSLOT = 1 + NUM_NEG

RB = 4096
PACK = 128 // DIM


def _repack_body(e_ref, o_ref, eo_ref, oo_ref):
    ey = e_ref[...].T.reshape(RB // PACK, PACK, DIM)
    oy = o_ref[...].T.reshape(RB // PACK, PACK, DIM)
    for a in range(PACK):
        eo_ref[:, pl.ds(a * DIM, DIM)] = ey[:, a, :]
        oo_ref[:, pl.ds(a * DIM, DIM)] = oy[:, a, :]


def _sc_body(inp_idx_hbm, tgt_idx_hbm, neg_idx_hbm, emb_hbm, outw_hbm,
             logits_hbm,
             idx_i_v, idx_t_v, idx_n_v, inp_rows, tgt_rows, neg_rows,
             logit_v, sem):
    wid = lax.axis_index("s") * NUM_CORES + lax.axis_index("c")

    pltpu.sync_copy(inp_idx_hbm.at[pl.ds(wid * BPW, BPW)], idx_i_v)
    pltpu.sync_copy(tgt_idx_hbm.at[pl.ds(wid * BPW, BPW)], idx_t_v)
    pltpu.sync_copy(neg_idx_hbm.at[pl.ds(wid * BPW * NUM_NEG, BPW * NUM_NEG)],
                    idx_n_v)

    copies = []
    for j in range(N_CHUNKS):
        c = pltpu.make_async_copy(emb_hbm.at[idx_i_v.at[pl.ds(j * CHUNK, CHUNK)]],
                                  inp_rows.at[pl.ds(j * CHUNK, CHUNK)], sem)
        c.start()
        copies.append(c)
    for j in range(N_CHUNKS):
        c = pltpu.make_async_copy(outw_hbm.at[idx_t_v.at[pl.ds(j * CHUNK, CHUNK)]],
                                  tgt_rows.at[pl.ds(j * CHUNK, CHUNK)], sem)
        c.start()
        copies.append(c)
    for j in range(N_NEG_CHUNKS):
        c = pltpu.make_async_copy(outw_hbm.at[idx_n_v.at[pl.ds(j * CHUNK, CHUNK)]],
                                  neg_rows.at[pl.ds(j * CHUNK, CHUNK)], sem)
        c.start()
        copies.append(c)
    for c in copies:
        c.wait()

    lane = lax.iota(jnp.int32, 16)
    zeros = jnp.zeros((16,), jnp.float32)

    def group_body(g, carry):
        b0 = g * 16
        rows16 = b0 + lane
        rows_neg = rows16 * NUM_NEG

        def d_body(d, accs):
            dcol = jnp.zeros((16,), jnp.int32) + d
            iv = plsc.load_gather(inp_rows, [rows16, dcol])
            tv = plsc.load_gather(tgt_rows, [rows16, dcol])
            new = [accs[0] - iv * tv]
            for k in range(NUM_NEG):
                nv = plsc.load_gather(neg_rows, [rows_neg + k, dcol])
                new.append(accs[k + 1] + iv * nv)
            return tuple(new)

        accs = lax.fori_loop(0, DIM, d_body, (zeros,) * NSLOT)
        for s in range(NSLOT):
            logit_v[pl.ds(s * BPW + b0, 16)] = accs[s]
        return carry

    lax.fori_loop(0, BPW // 16, group_body, 0)

    pltpu.sync_copy(logit_v, logits_hbm.at[pl.ds(wid * NSLOT * BPW,
                                                 NSLOT * BPW)])


def _tc_body(x_ref, o_ref):
    x = x_ref[...]
    o_ref[0, 0] = jnp.sum(jax.nn.softplus(x)) / BATCH


def kernel(input_idx, target_idx, neg_idx, emb_table, out_table):
    inp1d = input_idx.astype(jnp.int32).reshape(BATCH)
    tgt1d = target_idx.astype(jnp.int32).reshape(BATCH)
    neg1d = neg_idx.astype(jnp.int32).reshape(BATCH * NUM_NEG)

    n_blocks = (VOCAB + RB - 1) // RB
    emb_rm, out_rm = pl.pallas_call(
        _repack_body,
        grid=(n_blocks,),
        in_specs=[
            pl.BlockSpec((DIM, RB), lambda i: (0, i)),
            pl.BlockSpec((DIM, RB), lambda i: (0, i)),
        ],
        out_specs=[
            pl.BlockSpec((RB // PACK, 128), lambda i: (i, 0)),
            pl.BlockSpec((RB // PACK, 128), lambda i: (i, 0)),
        ],
        out_shape=[
            jax.ShapeDtypeStruct((VOCAB * DIM // 128, 128), jnp.float32),
            jax.ShapeDtypeStruct((VOCAB * DIM // 128, 128), jnp.float32),
        ],
    )(emb_table.T, out_table.T)
    emb_rm = emb_rm.reshape(VOCAB, DIM)
    out_rm = out_rm.reshape(VOCAB, DIM)

    mesh = plsc.VectorSubcoreMesh(core_axis_name="c", subcore_axis_name="s",
                                  num_cores=NUM_CORES,
                                  num_subcores=NUM_SUBCORES)
    sc_fn = pl.kernel(
        _sc_body,
        out_type=jax.ShapeDtypeStruct((NUM_WORKERS * NSLOT * BPW,),
                                      jnp.float32),
        mesh=mesh,
        compiler_params=pltpu.CompilerParams(needs_layout_passes=False,
                                             use_tc_tiling_on_sc=False),
        scratch_types=[
            pltpu.VMEM((BPW,), jnp.int32),
            pltpu.VMEM((BPW,), jnp.int32),
            pltpu.VMEM((BPW * NUM_NEG,), jnp.int32),
            pltpu.VMEM((BPW, DIM), jnp.float32),
            pltpu.VMEM((BPW, DIM), jnp.float32),
            pltpu.VMEM((BPW * NUM_NEG, DIM), jnp.float32),
            pltpu.VMEM((NSLOT * BPW,), jnp.float32),
            pltpu.SemaphoreType.DMA,
        ],
    )
    logits = sc_fn(inp1d, tgt1d, neg1d, emb_rm, out_rm)

    flat = logits.reshape(NUM_WORKERS * NSLOT * BPW // 128, 128)
    loss = pl.pallas_call(
        _tc_body,
        out_shape=jax.ShapeDtypeStruct((1, 1), jnp.float32),
        out_specs=pl.BlockSpec(memory_space=pltpu.SMEM),
    )(flat)
    return loss[0, 0]


# speedup vs baseline: 5.4701x; 4.6011x over previous
import jax
import jax.numpy as jnp
from jax import lax
from jax.experimental import pallas as pl
from jax.experimental.pallas import tpu as pltpu
from jax.experimental.pallas import tpu_sc as plsc

VOCAB = 1_000_000
DIM = 32
BATCH = 16384
NUM_NEG = 5

NUM_CORES = 2
NUM_SUBCORES = 16
NUM_WORKERS = NUM_CORES * NUM_SUBCORES
BPW = BATCH // NUM_WORKERS
NSLOT = 1 + NUM_NEG

RB = 32768
PACK = 8
WPR = DIM // 2


def _repack_body(e_ref, o_ref, eo_ref, oo_ref):
    SUB = RB // PACK
    HALF = DIM // 2

    def bf_bits(v):
        return (v + 0x7FFF + ((v >> 16) & 1)) >> 16

    for src, dst in ((e_ref, eo_ref), (o_ref, oo_ref)):
        x = src[...]
        xl = x[0:HALF, :]
        xh = x[HALF:DIM, :]
        ml = jnp.concatenate(
            [xl[:, a * SUB:(a + 1) * SUB] for a in range(PACK)], axis=0)
        mh = jnp.concatenate(
            [xh[:, a * SUB:(a + 1) * SUB] for a in range(PACK)], axis=0)
        yl = jax.lax.bitcast_convert_type(ml.T, jnp.int32)
        yh = jax.lax.bitcast_convert_type(mh.T, jnp.int32)
        dst[...] = (bf_bits(yh) << 16) | (bf_bits(yl) & 0xFFFF)


def _sc_body(inp_idx_hbm, tgt_idx_hbm, neg_idx_hbm, emb_hbm, outw_hbm,
             logits_hbm,
             idx_i_v, idx_t_v, idx_n_v, inp_rows, tgt_rows, neg_rows,
             logit_v, sem, sem_n):
    wid = lax.axis_index("s") * NUM_CORES + lax.axis_index("c")

    stage = [
        pltpu.make_async_copy(inp_idx_hbm.at[pl.ds(wid * BPW, BPW)],
                              idx_i_v, sem),
        pltpu.make_async_copy(tgt_idx_hbm.at[pl.ds(wid * BPW, BPW)],
                              idx_t_v, sem),
        pltpu.make_async_copy(
            neg_idx_hbm.at[pl.ds(wid * BPW * NUM_NEG, BPW * NUM_NEG)],
            idx_n_v, sem),
    ]
    for c in stage:
        c.start()
    for c in stage:
        c.wait()

    def remap(buf, n):
        def step(i, carry):
            r = buf[pl.ds(i * 16, 16)]
            m = (((r >> 15) << 15) | ((r & 4095) << 3) | ((r >> 12) & 7))
            buf[pl.ds(i * 16, 16)] = m
            return carry
        lax.fori_loop(0, n // 16, step, 0)

    copies = []
    for src, idx_v, dst, n, sm in ((emb_hbm, idx_i_v, inp_rows, BPW, sem),
                                   (outw_hbm, idx_t_v, tgt_rows, BPW, sem),
                                   (outw_hbm, idx_n_v, neg_rows,
                                    BPW * NUM_NEG, sem_n)):
        remap(idx_v, n)
        c = pltpu.make_async_copy(src.at[idx_v], dst, sm)
        c.start()
        copies.append(c)
    copies[0].wait()
    copies[1].wait()

    lane = lax.iota(jnp.int32, 16)
    zeros = jnp.zeros((16,), jnp.float32)

    def fetch2(buf, rows, pcol):
        wv = plsc.load_gather(buf, [rows, pcol])
        return plsc.unpack(plsc.bitcast(wv, jnp.bfloat16),
                           format=plsc.PackFormat.INTERLEAVED,
                           preferred_element_type=jnp.float32)

    def pos_body(g, carry):
        b0 = g * 16
        rows16 = b0 + lane

        def d_body(p4, acc):
            for u in range(4):
                pcol = jnp.zeros((16,), jnp.int32) + (p4 * 4 + u)
                ie, io = fetch2(inp_rows, rows16, pcol)
                te, to = fetch2(tgt_rows, rows16, pcol)
                acc = acc - ie * te - io * to
            return acc

        acc = lax.fori_loop(0, WPR // 4, d_body, zeros)
        logit_v[pl.ds(b0, 16)] = acc
        return carry

    lax.fori_loop(0, BPW // 16, pos_body, 0)
    copies[2].wait()

    def neg_body(g, carry):
        b0 = g * 16
        rows16 = b0 + lane
        rows_neg = rows16 * NUM_NEG

        def d_body(p4, accs):
            new = list(accs)
            for u in range(4):
                pcol = jnp.zeros((16,), jnp.int32) + (p4 * 4 + u)
                ie, io = fetch2(inp_rows, rows16, pcol)
                for k in range(NUM_NEG):
                    ne, no = fetch2(neg_rows, rows_neg + k, pcol)
                    new[k] = new[k] + ie * ne + io * no
            return tuple(new)

        accs = lax.fori_loop(0, WPR // 4, d_body, (zeros,) * NUM_NEG)
        for k in range(NUM_NEG):
            logit_v[pl.ds((k + 1) * BPW + b0, 16)] = accs[k]
        return carry

    lax.fori_loop(0, BPW // 16, neg_body, 0)

    pltpu.sync_copy(logit_v, logits_hbm.at[pl.ds(wid * NSLOT * BPW,
                                                 NSLOT * BPW)])


def _tc_body(x_ref, o_ref):
    x = x_ref[...]
    o_ref[0, 0] = jnp.sum(jax.nn.softplus(x)) / BATCH


def kernel(input_idx, target_idx, neg_idx, emb_table, out_table):
    inp1d = input_idx.astype(jnp.int32).reshape(BATCH)
    tgt1d = target_idx.astype(jnp.int32).reshape(BATCH)
    neg1d = neg_idx.astype(jnp.int32).reshape(BATCH * NUM_NEG)

    n_blocks = (VOCAB + RB - 1) // RB
    q_rows = n_blocks * RB // PACK
    vocab_pad = q_rows * 128 // WPR
    emb_rm, out_rm = pl.pallas_call(
        _repack_body,
        grid=(n_blocks,),
        in_specs=[
            pl.BlockSpec((DIM, RB), lambda i: (0, i)),
            pl.BlockSpec((DIM, RB), lambda i: (0, i)),
        ],
        out_specs=[
            pl.BlockSpec((RB // PACK, 128), lambda i: (i, 0)),
            pl.BlockSpec((RB // PACK, 128), lambda i: (i, 0)),
        ],
        out_shape=[
            jax.ShapeDtypeStruct((q_rows, 128), jnp.int32),
            jax.ShapeDtypeStruct((q_rows, 128), jnp.int32),
        ],
    )(emb_table.T, out_table.T)
    emb_rm = emb_rm.reshape(vocab_pad, WPR)
    out_rm = out_rm.reshape(vocab_pad, WPR)

    mesh = plsc.VectorSubcoreMesh(core_axis_name="c", subcore_axis_name="s",
                                  num_cores=NUM_CORES,
                                  num_subcores=NUM_SUBCORES)
    sc_fn = pl.kernel(
        _sc_body,
        out_type=jax.ShapeDtypeStruct((NUM_WORKERS * NSLOT * BPW,),
                                      jnp.float32),
        mesh=mesh,
        compiler_params=pltpu.CompilerParams(needs_layout_passes=False,
                                             use_tc_tiling_on_sc=False),
        scratch_types=[
            pltpu.VMEM((BPW,), jnp.int32),
            pltpu.VMEM((BPW,), jnp.int32),
            pltpu.VMEM((BPW * NUM_NEG,), jnp.int32),
            pltpu.VMEM((BPW, WPR), jnp.int32),
            pltpu.VMEM((BPW, WPR), jnp.int32),
            pltpu.VMEM((BPW * NUM_NEG, WPR), jnp.int32),
            pltpu.VMEM((NSLOT * BPW,), jnp.float32),
            pltpu.SemaphoreType.DMA,
            pltpu.SemaphoreType.DMA,
        ],
    )
    logits = sc_fn(inp1d, tgt1d, neg1d, emb_rm, out_rm)

    flat = logits.reshape(NUM_WORKERS * NSLOT * BPW // 128, 128)
    loss = pl.pallas_call(
        _tc_body,
        out_shape=jax.ShapeDtypeStruct((1, 1), jnp.float32),
        out_specs=pl.BlockSpec(memory_space=pltpu.SMEM),
    )(flat)
    return loss[0, 0]
